# baseline (device time: 28443 ns/iter reference)
import jax
import jax.numpy as jnp
from jax import lax
from jax.experimental import pallas as pl
from jax.experimental.pallas import tpu as pltpu

N_DEV = 4


def kernel(table, idx):
    v_per, d = table.shape
    n = idx.shape[0]
    h = n // 2
    vh = v_per // 2
    idx2 = idx.reshape(n, 1)

    def body(
        table_ref, idx_ref, out_ref,
        tvmem, acc_a, acc_b, rbuf, load_sems, send_sems, recv_sems,
    ):
        my = lax.axis_index("i")
        p_a = my ^ 1
        p_b = 3 - my

        loads = []
        for k in range(2):
            cp = pltpu.make_async_copy(
                table_ref.at[pl.ds(k * vh, vh), :],
                tvmem.at[k],
                load_sems.at[k],
            )
            cp.start()
            loads.append(cp)

        barrier_sem = pltpu.get_barrier_semaphore()
        for nbr in [p_a, p_b]:
            pl.semaphore_signal(
                barrier_sem, inc=1,
                device_id=(nbr,), device_id_type=pl.DeviceIdType.MESH,
            )
        pl.semaphore_wait(barrier_sem, 2)

        def exchange(src, ph, half, tgt):
            return pltpu.make_async_remote_copy(
                src_ref=src,
                dst_ref=rbuf.at[ph, half],
                send_sem=send_sems.at[ph, half],
                recv_sem=recv_sems.at[ph, half],
                device_id=(tgt,),
                device_id_type=pl.DeviceIdType.MESH,
            )

        local = idx_ref[...] - my * v_per
        iota = lax.broadcasted_iota(jnp.int32, (h, v_per), 1)

        loads[0].wait()
        t0 = tvmem[0].astype(jnp.bfloat16)
        pa0 = jnp.dot(
            (iota[:, :vh] == local[:h]).astype(jnp.bfloat16),
            t0, preferred_element_type=jnp.float32,
        )
        loads[1].wait()
        t1 = tvmem[1].astype(jnp.bfloat16)
        pa1 = jnp.dot(
            (iota[:, vh:] == local[:h]).astype(jnp.bfloat16),
            t1, preferred_element_type=jnp.float32,
        )
        acc_a[...] = (pa0 + pa1).astype(jnp.bfloat16)
        a0 = exchange(acc_a, 0, 0, p_a)
        a0.start()

        pb0 = jnp.dot(
            (iota[:, :vh] == local[h:]).astype(jnp.bfloat16),
            t0, preferred_element_type=jnp.float32,
        )
        pb1 = jnp.dot(
            (iota[:, vh:] == local[h:]).astype(jnp.bfloat16),
            t1, preferred_element_type=jnp.float32,
        )
        acc_b[...] = (pb0 + pb1).astype(jnp.bfloat16)
        b0 = exchange(acc_b, 0, 1, p_b)
        b0.start()

        a0.wait()
        acc_a[...] += rbuf[0, 0]
        a1 = exchange(acc_a, 1, 0, p_b)
        a1.start()

        b0.wait()
        acc_b[...] += rbuf[0, 1]
        b1 = exchange(acc_b, 1, 1, p_a)
        b1.start()

        a1.wait()
        out_ref[:h, :] = acc_a[...] + rbuf[1, 0]
        b1.wait()
        out_ref[h:, :] = acc_b[...] + rbuf[1, 1]

    return pl.pallas_call(
        body,
        out_shape=jax.ShapeDtypeStruct((n, d), jnp.bfloat16),
        in_specs=[
            pl.BlockSpec(memory_space=pltpu.MemorySpace.HBM),
            pl.BlockSpec(memory_space=pltpu.VMEM),
        ],
        out_specs=pl.BlockSpec(memory_space=pltpu.VMEM),
        scratch_shapes=[
            pltpu.VMEM((2, vh, d), jnp.float32),
            pltpu.VMEM((h, d), jnp.bfloat16),
            pltpu.VMEM((h, d), jnp.bfloat16),
            pltpu.VMEM((2, 2, h, d), jnp.bfloat16),
            pltpu.SemaphoreType.DMA((2,)),
            pltpu.SemaphoreType.DMA((2, 2)),
            pltpu.SemaphoreType.DMA((2, 2)),
        ],
        compiler_params=pltpu.CompilerParams(collective_id=0),
    )(table, idx2)


# device time: 27383 ns/iter; 1.0387x vs baseline; 1.0387x over previous
import jax
import jax.numpy as jnp
from jax import lax
from jax.experimental import pallas as pl
from jax.experimental.pallas import tpu as pltpu

N_DEV = 4


def kernel(table, idx):
    v_per, d = table.shape
    n = idx.shape[0]
    h = n // 2

    def body(
        table_ref, idx_ref, out_ref,
        acc_a, acc_b, obuf, rbuf, out_sems, send_sems, recv_sems,
    ):
        my = lax.axis_index("i")
        p_a = my ^ 1
        p_b = 3 - my

        barrier_sem = pltpu.get_barrier_semaphore()
        for nbr in [p_a, p_b]:
            pl.semaphore_signal(
                barrier_sem, inc=1,
                device_id=(nbr,), device_id_type=pl.DeviceIdType.MESH,
            )
        pl.semaphore_wait(barrier_sem, 2)

        def exchange(src, ph, half, tgt):
            return pltpu.make_async_remote_copy(
                src_ref=src,
                dst_ref=rbuf.at[ph, half],
                send_sem=send_sems.at[ph, half],
                recv_sem=recv_sems.at[ph, half],
                device_id=(tgt,),
                device_id_type=pl.DeviceIdType.MESH,
            )

        local = jnp.reshape(idx_ref[...], (n, 1)) - my * v_per
        tb = table_ref[...].astype(jnp.bfloat16)
        iota = lax.broadcasted_iota(jnp.int32, (h, v_per), 1)

        onehot_a = (iota == local[:h]).astype(jnp.bfloat16)
        acc_a[...] = jnp.dot(
            onehot_a, tb, preferred_element_type=jnp.float32
        ).astype(jnp.bfloat16)
        a0 = exchange(acc_a, 0, 0, p_a)
        a0.start()

        onehot_b = (iota == local[h:]).astype(jnp.bfloat16)
        acc_b[...] = jnp.dot(
            onehot_b, tb, preferred_element_type=jnp.float32
        ).astype(jnp.bfloat16)
        b0 = exchange(acc_b, 0, 1, p_b)
        b0.start()

        a0.wait()
        acc_a[...] += rbuf[0, 0]
        a1 = exchange(acc_a, 1, 0, p_b)
        a1.start()

        b0.wait()
        acc_b[...] += rbuf[0, 1]
        b1 = exchange(acc_b, 1, 1, p_a)
        b1.start()

        a1.wait()
        obuf[:h, :] = acc_a[...] + rbuf[1, 0]
        oc_a = pltpu.make_async_copy(
            obuf.at[pl.ds(0, h), :], out_ref.at[pl.ds(0, h), :],
            out_sems.at[0],
        )
        oc_a.start()

        b1.wait()
        obuf[h:, :] = acc_b[...] + rbuf[1, 1]
        oc_b = pltpu.make_async_copy(
            obuf.at[pl.ds(h, h), :], out_ref.at[pl.ds(h, h), :],
            out_sems.at[1],
        )
        oc_b.start()
        oc_a.wait()
        oc_b.wait()

    return pl.pallas_call(
        body,
        out_shape=jax.ShapeDtypeStruct((n, d), jnp.bfloat16),
        in_specs=[
            pl.BlockSpec(memory_space=pltpu.VMEM),
            pl.BlockSpec(memory_space=pltpu.VMEM),
        ],
        out_specs=pl.BlockSpec(memory_space=pl.ANY),
        scratch_shapes=[
            pltpu.VMEM((h, d), jnp.bfloat16),
            pltpu.VMEM((h, d), jnp.bfloat16),
            pltpu.VMEM((n, d), jnp.bfloat16),
            pltpu.VMEM((2, 2, h, d), jnp.bfloat16),
            pltpu.SemaphoreType.DMA((2,)),
            pltpu.SemaphoreType.DMA((2, 2)),
            pltpu.SemaphoreType.DMA((2, 2)),
        ],
        compiler_params=pltpu.CompilerParams(collective_id=0),
    )(table, idx)


# device time: 26050 ns/iter; 1.0919x vs baseline; 1.0512x over previous
import jax
import jax.numpy as jnp
from jax import lax
from jax.experimental import pallas as pl
from jax.experimental.pallas import tpu as pltpu

N_DEV = 4
C = 8
LAG = 2


def kernel(table, idx):
    v_per, d = table.shape
    n = idx.shape[0]
    rows = n // C

    def body(
        table_ref, idx_ref, out_ref,
        acc, obuf, rbuf, out_sems, send_sems, recv_sems,
    ):
        my = lax.axis_index("i")
        p_a = my ^ 1
        p_b = 3 - my

        def exchange(c, stage, tgt):
            return pltpu.make_async_remote_copy(
                src_ref=acc.at[c],
                dst_ref=rbuf.at[stage, c],
                send_sem=send_sems.at[stage, c],
                recv_sem=recv_sems.at[stage, c],
                device_id=(tgt,),
                device_id_type=pl.DeviceIdType.MESH,
            )

        def partners(c):
            return (p_a, p_b) if c % 2 == 0 else (p_b, p_a)

        s0 = [None] * C
        s1 = [None] * C
        ocs = [None] * C
        tb = table_ref[...].astype(jnp.bfloat16)
        local = jnp.reshape(idx_ref[...], (n, 1)) - my * v_per
        iota = lax.broadcasted_iota(jnp.int32, (rows, v_per), 1)

        def drain_s0(c):
            s0[c].wait()
            acc[c] += rbuf[0, c]
            s1[c] = exchange(c, 1, partners(c)[1])
            s1[c].start()

        def drain_s1(c):
            s1[c].wait()
            obuf[c] = acc[c] + rbuf[1, c]
            ocs[c] = pltpu.make_async_copy(
                obuf.at[c], out_ref.at[pl.ds(c * rows, rows), :],
                out_sems.at[c],
            )
            ocs[c].start()

        for c in range(C):
            onehot = (iota == local[c * rows:(c + 1) * rows]).astype(
                jnp.bfloat16
            )
            acc[c] = jnp.dot(
                onehot, tb, preferred_element_type=jnp.float32
            ).astype(jnp.bfloat16)
            if c == 0:
                barrier_sem = pltpu.get_barrier_semaphore()
                for nbr in [p_a, p_b]:
                    pl.semaphore_signal(
                        barrier_sem, inc=1,
                        device_id=(nbr,),
                        device_id_type=pl.DeviceIdType.MESH,
                    )
                pl.semaphore_wait(barrier_sem, 2)
            s0[c] = exchange(c, 0, partners(c)[0])
            s0[c].start()
            if c >= LAG:
                drain_s0(c - LAG)
            if c >= 2 * LAG:
                drain_s1(c - 2 * LAG)
        for c in range(C - LAG, C):
            drain_s0(c)
        for c in range(C - 2 * LAG, C):
            drain_s1(c)
        for c in range(C):
            ocs[c].wait()

    return pl.pallas_call(
        body,
        out_shape=jax.ShapeDtypeStruct((n, d), jnp.bfloat16),
        in_specs=[
            pl.BlockSpec(memory_space=pltpu.VMEM),
            pl.BlockSpec(memory_space=pltpu.VMEM),
        ],
        out_specs=pl.BlockSpec(memory_space=pl.ANY),
        scratch_shapes=[
            pltpu.VMEM((C, rows, d), jnp.bfloat16),
            pltpu.VMEM((C, rows, d), jnp.bfloat16),
            pltpu.VMEM((2, C, rows, d), jnp.bfloat16),
            pltpu.SemaphoreType.DMA((C,)),
            pltpu.SemaphoreType.DMA((2, C)),
            pltpu.SemaphoreType.DMA((2, C)),
        ],
        compiler_params=pltpu.CompilerParams(collective_id=0),
    )(table, idx)
